# baseline (speedup 1.0000x reference)
import functools

import jax
import jax.numpy as jnp
from jax import lax
from jax.experimental import pallas as pl
from jax.experimental.pallas import tpu as pltpu
from jax.experimental.pallas import tpu_sc as plsc

RAYS = 65536
NC = 64
NF = 128
NB = NC - 1
NW = NC - 2
NOUT = NC + NF
NSORT = 256
C = 128

NWORK = 32
RW = RAYS // NWORK
CH = 16
HLEN = 144
OFF = 8


def _sc_sample_kernel(depth_hbm, w_hbm, u_hbm, out_hbm,
                      dbuf, wbuf, sbuf, ubuf,
                      fv, mv, sl, al, ha, hb, dscr):
    widx = lax.axis_index("s") * 2 + lax.axis_index("c")
    pltpu.sync_copy(u_hbm, ubuf)
    iot = lax.iota(jnp.int32, 16)
    zero = jnp.zeros((16,), jnp.float32)
    m_lo = iot >= 1
    m_hi = iot <= 14

    def ray_body(r, _):
        wv = [wbuf[r, pl.ds(16 * v, 16)] + 1e-5 for v in range(4)]
        wv[0] = jnp.where(m_lo, wv[0], zero)
        wv[3] = jnp.where(m_hi, wv[3], zero)
        sums = [jnp.sum(x) for x in wv]
        inv_v = 1.0 / jnp.full((16,), sums[0] + sums[1] + sums[2] + sums[3],
                               jnp.float32)
        carry = zero
        for v in range(4):
            pdfv = wv[v] * inv_v
            fv[pl.ds(OFF + 16 * v, 16)] = plsc.cumsum(pdfv) + carry
            carry = carry + jnp.full((16,), sums[v], jnp.float32) * inv_v
        fv[pl.ds(OFF + 64, 16)] = zero

        d3 = dbuf[r, pl.ds(48, 16)]
        dscr[pl.ds(0, 16)] = d3
        for v in range(3):
            mv[pl.ds(OFF + 16 * v, 16)] = 0.5 * (
                dbuf[r, pl.ds(16 * v, 16)] + dbuf[r, pl.ds(16 * v + 1, 16)])
        mv[pl.ds(OFF + 48, 16)] = 0.5 * (d3 + dscr[pl.ds(1, 16)])

        for v in range(4):
            f0 = fv[pl.ds(OFF + 16 * v, 16)]
            fd = fv[pl.ds(OFF + 16 * v + 1, 16)] - f0
            den = jnp.where(fd < 1e-5, jnp.full((16,), 1.0, jnp.float32), fd)
            m0 = mv[pl.ds(OFF + 16 * v, 16)]
            s = (mv[pl.ds(OFF + 16 * v + 1, 16)] - m0) / den
            if v == 3:
                s = jnp.where(iot <= 13, s, zero)
            sl[pl.ds(OFF + 16 * v, 16)] = s
            al[pl.ds(OFF + 16 * v, 16)] = m0 - f0 * s

        for q in range(HLEN // 16):
            ha[pl.ds(16 * q, 16)] = zero
            hb[pl.ds(16 * q, 16)] = zero
        for v in range(4):
            f0 = fv[pl.ds(OFF + 16 * v, 16)]
            tf = f0 * 127.0
            ti = tf.astype(jnp.int32)
            t = jnp.minimum(ti + (ti.astype(jnp.float32) < tf), 128)
            da = al[pl.ds(OFF + 16 * v, 16)] - al[pl.ds(OFF + 16 * v - 1, 16)]
            ds_ = sl[pl.ds(OFF + 16 * v, 16)] - sl[pl.ds(OFF + 16 * v - 1, 16)]
            mask = m_lo if v == 0 else (m_hi if v == 3 else None)
            plsc.addupdate_scatter(ha, [t], da, mask=mask)
            plsc.addupdate_scatter(hb, [t], ds_, mask=mask)

        a0 = al[pl.ds(OFF, 16)][0]
        s0 = sl[pl.ds(OFF, 16)][0]
        ca = 0.0
        cb = 0.0
        for q in range(8):
            hav = ha[pl.ds(16 * q, 16)]
            hbv = hb[pl.ds(16 * q, 16)]
            acc_a = plsc.cumsum(hav) + (ca + a0)
            acc_b = plsc.cumsum(hbv) + (cb + s0)
            uq = ubuf[pl.ds(16 * q, 16)]
            sbuf[r, pl.ds(16 * q, 16)] = acc_a + uq * acc_b
            ca = ca + jnp.sum(hav)
            cb = cb + jnp.sum(hbv)
        return 0

    def chunk_body(g, _):
        base = widx * RW + g * CH
        pltpu.sync_copy(depth_hbm.at[pl.ds(base, CH), :], dbuf)
        pltpu.sync_copy(w_hbm.at[pl.ds(base, CH), :], wbuf)
        lax.fori_loop(0, CH, ray_body, 0, unroll=False)
        pltpu.sync_copy(sbuf, out_hbm.at[pl.ds(base, CH), :])
        return 0

    lax.fori_loop(0, RW // CH, chunk_body, 0, unroll=False)


@jax.jit
def _sc_sample(depth, weights, u):
    kfn = pl.kernel(
        _sc_sample_kernel,
        out_type=jax.ShapeDtypeStruct((RAYS, NF), jnp.float32),
        mesh=plsc.VectorSubcoreMesh(core_axis_name="c", subcore_axis_name="s"),
        compiler_params=pltpu.CompilerParams(needs_layout_passes=False),
        scratch_types=[
            pltpu.VMEM((CH, NC), jnp.float32),
            pltpu.VMEM((CH, NC), jnp.float32),
            pltpu.VMEM((CH, NF), jnp.float32),
            pltpu.VMEM((NF,), jnp.float32),
            pltpu.VMEM((96,), jnp.float32),
            pltpu.VMEM((96,), jnp.float32),
            pltpu.VMEM((96,), jnp.float32),
            pltpu.VMEM((96,), jnp.float32),
            pltpu.VMEM((HLEN,), jnp.float32),
            pltpu.VMEM((HLEN,), jnp.float32),
            pltpu.VMEM((24,), jnp.float32),
        ],
    )
    return kfn(depth, weights, u)


def _substage(x, nrows, j, k, descending=False):
    m = nrows // (2 * j)
    y = x.reshape(m, 2 * j, C)
    a = y[:, :j, :]
    b = y[:, j:, :]
    lo = jnp.minimum(a, b)
    hi = jnp.maximum(a, b)
    if k >= nrows and not descending:
        na, nb = lo, hi
    elif k >= nrows:
        na, nb = hi, lo
    else:
        blk = jax.lax.broadcasted_iota(jnp.int32, (m, 1, C), 0)
        asc = ((blk * (2 * j)) & k) == 0
        if descending:
            asc = jnp.logical_not(asc)
        na = jnp.where(asc, lo, hi)
        nb = jnp.where(asc, hi, lo)
    return jnp.concatenate([na, nb], axis=1).reshape(nrows, C)


def _bitonic_sort(x, nrows, descending=False):
    k = 2
    while k <= nrows:
        j = k // 2
        while j >= 1:
            x = _substage(x, nrows, j, k, descending)
            j //= 2
        k *= 2
    return x


def _sort_body(d_ref, s_ref, o_ref):
    d = d_ref[...]
    samples = s_ref[...]
    s_sorted = _bitonic_sort(samples, NF, descending=False)
    d_sorted = _bitonic_sort(d, NC, descending=True)
    x = jnp.concatenate(
        [s_sorted, jnp.full((NSORT - NOUT, C), jnp.inf, jnp.float32),
         d_sorted], axis=0)
    j = NSORT // 2
    while j >= 1:
        x = _substage(x, NSORT, j, NSORT)
        j //= 2
    o_ref[...] = x[:NOUT, :]


@jax.jit
def _tc_sort(depth_t, samples_t):
    grid = RAYS // C
    return pl.pallas_call(
        _sort_body,
        grid=(grid,),
        in_specs=[
            pl.BlockSpec((NC, C), lambda i: (0, i)),
            pl.BlockSpec((NF, C), lambda i: (0, i)),
        ],
        out_specs=pl.BlockSpec((NOUT, C), lambda i: (0, i)),
        out_shape=jax.ShapeDtypeStruct((NOUT, RAYS), jnp.float32),
    )(depth_t, samples_t)


def kernel(depth_rays_values_coarse, coarse_weights, perturb):
    del perturb
    u = jnp.linspace(0.0, 1.0, NF, dtype=jnp.float32)
    samples = _sc_sample(depth_rays_values_coarse, coarse_weights, u)
    out_t = _tc_sort(depth_rays_values_coarse.T, samples.T)
    return out_t.T


# speedup vs baseline: 1.0770x; 1.0770x over previous
import functools

import jax
import jax.numpy as jnp
from jax import lax
from jax.experimental import pallas as pl
from jax.experimental.pallas import tpu as pltpu
from jax.experimental.pallas import tpu_sc as plsc

RAYS = 65536
NC = 64
NF = 128
NB = NC - 1
NW = NC - 2
NOUT = NC + NF
NSORT = 256
C = 128

NWORK = 32
RW = RAYS // NWORK
CH = 64
HLEN = 144
OFF = 8


def _sc_sample_kernel(depth_hbm, w_hbm, u_hbm, out_hbm,
                      dbuf, wbuf, sbuf, ubuf,
                      fv, mv, sl, al, ha, hb, dscr):
    widx = lax.axis_index("s") * 2 + lax.axis_index("c")
    pltpu.sync_copy(u_hbm, ubuf)
    iot = lax.iota(jnp.int32, 16)
    zero = jnp.zeros((16,), jnp.float32)
    fv[pl.ds(OFF + 64, 16)] = zero
    m_lo = iot >= 1
    m_hi = iot <= 14

    def ray_body(r, _):
        wv = [wbuf[r, pl.ds(16 * v, 16)] + 1e-5 for v in range(4)]
        wv[0] = jnp.where(m_lo, wv[0], zero)
        wv[3] = jnp.where(m_hi, wv[3], zero)
        sums = [jnp.sum(x) for x in wv]
        inv_v = 1.0 / jnp.full((16,), sums[0] + sums[1] + sums[2] + sums[3],
                               jnp.float32)
        carry = zero
        for v in range(4):
            pdfv = wv[v] * inv_v
            fv[pl.ds(OFF + 16 * v, 16)] = plsc.cumsum(pdfv) + carry
            carry = carry + jnp.full((16,), sums[v], jnp.float32) * inv_v

        d3 = dbuf[r, pl.ds(48, 16)]
        dscr[pl.ds(0, 16)] = d3
        for v in range(3):
            mv[pl.ds(OFF + 16 * v, 16)] = 0.5 * (
                dbuf[r, pl.ds(16 * v, 16)] + dbuf[r, pl.ds(16 * v + 1, 16)])
        mv[pl.ds(OFF + 48, 16)] = 0.5 * (d3 + dscr[pl.ds(1, 16)])

        for v in range(4):
            f0 = fv[pl.ds(OFF + 16 * v, 16)]
            fd = fv[pl.ds(OFF + 16 * v + 1, 16)] - f0
            den = jnp.where(fd < 1e-5, jnp.full((16,), 1.0, jnp.float32), fd)
            m0 = mv[pl.ds(OFF + 16 * v, 16)]
            s = (mv[pl.ds(OFF + 16 * v + 1, 16)] - m0) / den
            if v == 3:
                s = jnp.where(iot <= 13, s, zero)
            sl[pl.ds(OFF + 16 * v, 16)] = s
            al[pl.ds(OFF + 16 * v, 16)] = m0 - f0 * s

        for q in range(HLEN // 16):
            ha[pl.ds(16 * q, 16)] = zero
            hb[pl.ds(16 * q, 16)] = zero
        for v in range(4):
            f0 = fv[pl.ds(OFF + 16 * v, 16)]
            tf = f0 * 127.0
            ti = tf.astype(jnp.int32)
            t = jnp.minimum(ti + (ti.astype(jnp.float32) < tf), 128)
            da = al[pl.ds(OFF + 16 * v, 16)] - al[pl.ds(OFF + 16 * v - 1, 16)]
            ds_ = sl[pl.ds(OFF + 16 * v, 16)] - sl[pl.ds(OFF + 16 * v - 1, 16)]
            mask = m_lo if v == 0 else (m_hi if v == 3 else None)
            plsc.addupdate_scatter(ha, [t], da, mask=mask)
            plsc.addupdate_scatter(hb, [t], ds_, mask=mask)

        a0 = al[pl.ds(OFF, 16)][0]
        s0 = sl[pl.ds(OFF, 16)][0]
        ca = 0.0
        cb = 0.0
        for q in range(8):
            hav = ha[pl.ds(16 * q, 16)]
            hbv = hb[pl.ds(16 * q, 16)]
            acc_a = plsc.cumsum(hav) + (ca + a0)
            acc_b = plsc.cumsum(hbv) + (cb + s0)
            uq = ubuf[pl.ds(16 * q, 16)]
            sbuf[r, pl.ds(16 * q, 16)] = acc_a + uq * acc_b
            ca = ca + jnp.sum(hav)
            cb = cb + jnp.sum(hbv)
        return 0

    def chunk_body(g, _):
        base = widx * RW + g * CH
        pltpu.sync_copy(depth_hbm.at[pl.ds(base, CH), :], dbuf)
        pltpu.sync_copy(w_hbm.at[pl.ds(base, CH), :], wbuf)
        lax.fori_loop(0, CH, ray_body, 0, unroll=2)
        pltpu.sync_copy(sbuf, out_hbm.at[pl.ds(base, CH), :])
        return 0

    lax.fori_loop(0, RW // CH, chunk_body, 0, unroll=False)


@jax.jit
def _sc_sample(depth, weights, u):
    kfn = pl.kernel(
        _sc_sample_kernel,
        out_type=jax.ShapeDtypeStruct((RAYS, NF), jnp.float32),
        mesh=plsc.VectorSubcoreMesh(core_axis_name="c", subcore_axis_name="s"),
        compiler_params=pltpu.CompilerParams(needs_layout_passes=False),
        scratch_types=[
            pltpu.VMEM((CH, NC), jnp.float32),
            pltpu.VMEM((CH, NC), jnp.float32),
            pltpu.VMEM((CH, NF), jnp.float32),
            pltpu.VMEM((NF,), jnp.float32),
            pltpu.VMEM((96,), jnp.float32),
            pltpu.VMEM((96,), jnp.float32),
            pltpu.VMEM((96,), jnp.float32),
            pltpu.VMEM((96,), jnp.float32),
            pltpu.VMEM((HLEN,), jnp.float32),
            pltpu.VMEM((HLEN,), jnp.float32),
            pltpu.VMEM((24,), jnp.float32),
        ],
    )
    return kfn(depth, weights, u)


def _substage(x, nrows, j, k, descending=False):
    m = nrows // (2 * j)
    y = x.reshape(m, 2 * j, C)
    a = y[:, :j, :]
    b = y[:, j:, :]
    lo = jnp.minimum(a, b)
    hi = jnp.maximum(a, b)
    if k >= nrows and not descending:
        na, nb = lo, hi
    elif k >= nrows:
        na, nb = hi, lo
    else:
        blk = jax.lax.broadcasted_iota(jnp.int32, (m, 1, C), 0)
        asc = ((blk * (2 * j)) & k) == 0
        if descending:
            asc = jnp.logical_not(asc)
        na = jnp.where(asc, lo, hi)
        nb = jnp.where(asc, hi, lo)
    return jnp.concatenate([na, nb], axis=1).reshape(nrows, C)


def _bitonic_sort(x, nrows, descending=False):
    k = 2
    while k <= nrows:
        j = k // 2
        while j >= 1:
            x = _substage(x, nrows, j, k, descending)
            j //= 2
        k *= 2
    return x


def _sort_body(d_ref, s_ref, o_ref):
    d = d_ref[...]
    samples = s_ref[...]
    s_sorted = _bitonic_sort(samples, NF, descending=False)
    d_sorted = _bitonic_sort(d, NC, descending=True)
    x = jnp.concatenate(
        [s_sorted, jnp.full((NSORT - NOUT, C), jnp.inf, jnp.float32),
         d_sorted], axis=0)
    j = NSORT // 2
    while j >= 1:
        x = _substage(x, NSORT, j, NSORT)
        j //= 2
    o_ref[...] = x[:NOUT, :]


@jax.jit
def _tc_sort(depth_t, samples_t):
    grid = RAYS // C
    return pl.pallas_call(
        _sort_body,
        grid=(grid,),
        in_specs=[
            pl.BlockSpec((NC, C), lambda i: (0, i)),
            pl.BlockSpec((NF, C), lambda i: (0, i)),
        ],
        out_specs=pl.BlockSpec((NOUT, C), lambda i: (0, i)),
        out_shape=jax.ShapeDtypeStruct((NOUT, RAYS), jnp.float32),
    )(depth_t, samples_t)


def kernel(depth_rays_values_coarse, coarse_weights, perturb):
    del perturb
    u = jnp.linspace(0.0, 1.0, NF, dtype=jnp.float32)
    samples = _sc_sample(depth_rays_values_coarse, coarse_weights, u)
    out_t = _tc_sort(depth_rays_values_coarse.T, samples.T)
    return out_t.T


# speedup vs baseline: 1.4855x; 1.3792x over previous
import functools

import jax
import jax.numpy as jnp
from jax import lax
from jax.experimental import pallas as pl
from jax.experimental.pallas import tpu as pltpu
from jax.experimental.pallas import tpu_sc as plsc

RAYS = 65536
NC = 64
NF = 128
NB = NC - 1
NW = NC - 2
NOUT = NC + NF
NSORT = 256
C = 128

NWORK = 32
RW = RAYS // NWORK
CH = 64
HLEN = 144
OFF = 8


def _sc_sample_kernel(depth_hbm, w_hbm, u_hbm, out_hbm,
                      dbuf, wbuf, sbuf, ubuf,
                      fv, mv, sl, al, ha, hb, dscr):
    widx = lax.axis_index("s") * 2 + lax.axis_index("c")
    pltpu.sync_copy(u_hbm, ubuf)
    iot = lax.iota(jnp.int32, 16)
    zero = jnp.zeros((16,), jnp.float32)
    fv[pl.ds(OFF + 64, 16)] = zero
    m_lo = iot >= 1
    m_hi = iot <= 14

    def ray_body(r, _):
        wv = [wbuf[r, pl.ds(16 * v, 16)] + 1e-5 for v in range(4)]
        wv[0] = jnp.where(m_lo, wv[0], zero)
        wv[3] = jnp.where(m_hi, wv[3], zero)
        sums = [jnp.sum(x) for x in wv]
        inv_v = 1.0 / jnp.full((16,), sums[0] + sums[1] + sums[2] + sums[3],
                               jnp.float32)
        carry = zero
        for v in range(4):
            pdfv = wv[v] * inv_v
            fv[pl.ds(OFF + 16 * v, 16)] = plsc.cumsum(pdfv) + carry
            carry = carry + jnp.full((16,), sums[v], jnp.float32) * inv_v

        d3 = dbuf[r, pl.ds(48, 16)]
        dscr[pl.ds(0, 16)] = d3
        for v in range(3):
            mv[pl.ds(OFF + 16 * v, 16)] = 0.5 * (
                dbuf[r, pl.ds(16 * v, 16)] + dbuf[r, pl.ds(16 * v + 1, 16)])
        mv[pl.ds(OFF + 48, 16)] = 0.5 * (d3 + dscr[pl.ds(1, 16)])

        for v in range(4):
            f0 = fv[pl.ds(OFF + 16 * v, 16)]
            fd = fv[pl.ds(OFF + 16 * v + 1, 16)] - f0
            den = jnp.where(fd < 1e-5, jnp.full((16,), 1.0, jnp.float32), fd)
            m0 = mv[pl.ds(OFF + 16 * v, 16)]
            s = (mv[pl.ds(OFF + 16 * v + 1, 16)] - m0) / den
            if v == 3:
                s = jnp.where(iot <= 13, s, zero)
            sl[pl.ds(OFF + 16 * v, 16)] = s
            al[pl.ds(OFF + 16 * v, 16)] = m0 - f0 * s

        for q in range(HLEN // 16):
            ha[pl.ds(16 * q, 16)] = zero
            hb[pl.ds(16 * q, 16)] = zero
        for v in range(4):
            f0 = fv[pl.ds(OFF + 16 * v, 16)]
            tf = f0 * 127.0
            ti = tf.astype(jnp.int32)
            t = jnp.minimum(ti + (ti.astype(jnp.float32) < tf), 128)
            da = al[pl.ds(OFF + 16 * v, 16)] - al[pl.ds(OFF + 16 * v - 1, 16)]
            ds_ = sl[pl.ds(OFF + 16 * v, 16)] - sl[pl.ds(OFF + 16 * v - 1, 16)]
            mask = m_lo if v == 0 else (m_hi if v == 3 else None)
            plsc.addupdate_scatter(ha, [t], da, mask=mask)
            plsc.addupdate_scatter(hb, [t], ds_, mask=mask)

        a0 = al[pl.ds(OFF, 16)][0]
        s0 = sl[pl.ds(OFF, 16)][0]
        ca = 0.0
        cb = 0.0
        for q in range(8):
            hav = ha[pl.ds(16 * q, 16)]
            hbv = hb[pl.ds(16 * q, 16)]
            acc_a = plsc.cumsum(hav) + (ca + a0)
            acc_b = plsc.cumsum(hbv) + (cb + s0)
            uq = ubuf[pl.ds(16 * q, 16)]
            sbuf[r, pl.ds(16 * q, 16)] = acc_a + uq * acc_b
            ca = ca + jnp.sum(hav)
            cb = cb + jnp.sum(hbv)
        return 0

    def chunk_body(g, _):
        base = widx * (depth_hbm.shape[0] // NWORK) + g * CH
        pltpu.sync_copy(depth_hbm.at[pl.ds(base, CH), :], dbuf)
        pltpu.sync_copy(w_hbm.at[pl.ds(base, CH), :], wbuf)
        lax.fori_loop(0, CH, ray_body, 0, unroll=2)
        pltpu.sync_copy(sbuf, out_hbm.at[pl.ds(base, CH), :])
        return 0

    lax.fori_loop(0, depth_hbm.shape[0] // NWORK // CH, chunk_body, 0,
                  unroll=False)


def _sc_sample(depth, weights, u):
    kfn = pl.kernel(
        _sc_sample_kernel,
        out_type=jax.ShapeDtypeStruct((depth.shape[0], NF), jnp.float32),
        mesh=plsc.VectorSubcoreMesh(core_axis_name="c", subcore_axis_name="s"),
        compiler_params=pltpu.CompilerParams(needs_layout_passes=False),
        scratch_types=[
            pltpu.VMEM((CH, NC), jnp.float32),
            pltpu.VMEM((CH, NC), jnp.float32),
            pltpu.VMEM((CH, NF), jnp.float32),
            pltpu.VMEM((NF,), jnp.float32),
            pltpu.VMEM((96,), jnp.float32),
            pltpu.VMEM((96,), jnp.float32),
            pltpu.VMEM((96,), jnp.float32),
            pltpu.VMEM((96,), jnp.float32),
            pltpu.VMEM((HLEN,), jnp.float32),
            pltpu.VMEM((HLEN,), jnp.float32),
            pltpu.VMEM((24,), jnp.float32),
        ],
    )
    return kfn(depth, weights, u)


def _substage(x, nrows, j, k, descending=False):
    m = nrows // (2 * j)
    y = x.reshape(m, 2 * j, C)
    a = y[:, :j, :]
    b = y[:, j:, :]
    lo = jnp.minimum(a, b)
    hi = jnp.maximum(a, b)
    if k >= nrows and not descending:
        na, nb = lo, hi
    elif k >= nrows:
        na, nb = hi, lo
    else:
        blk = jax.lax.broadcasted_iota(jnp.int32, (m, 1, C), 0)
        asc = ((blk * (2 * j)) & k) == 0
        if descending:
            asc = jnp.logical_not(asc)
        na = jnp.where(asc, lo, hi)
        nb = jnp.where(asc, hi, lo)
    return jnp.concatenate([na, nb], axis=1).reshape(nrows, C)


def _bitonic_sort(x, nrows, descending=False):
    k = 2
    while k <= nrows:
        j = k // 2
        while j >= 1:
            x = _substage(x, nrows, j, k, descending)
            j //= 2
        k *= 2
    return x


def _sort_body(d_ref, s_ref, o_ref):
    d = d_ref[...]
    samples = s_ref[...]
    s_sorted = _bitonic_sort(samples, NF, descending=False)
    d_sorted = _bitonic_sort(d, NC, descending=True)
    x = jnp.concatenate(
        [s_sorted, jnp.full((NSORT - NOUT, C), jnp.inf, jnp.float32),
         d_sorted], axis=0)
    j = NSORT // 2
    while j >= 1:
        x = _substage(x, NSORT, j, NSORT)
        j //= 2
    o_ref[...] = x[:NOUT, :]


def _tc_sort(depth_t, samples_t):
    nrays = depth_t.shape[1]
    return pl.pallas_call(
        _sort_body,
        grid=(nrays // C,),
        in_specs=[
            pl.BlockSpec((NC, C), lambda i: (0, i)),
            pl.BlockSpec((NF, C), lambda i: (0, i)),
        ],
        out_specs=pl.BlockSpec((NOUT, C), lambda i: (0, i)),
        out_shape=jax.ShapeDtypeStruct((NOUT, nrays), jnp.float32),
    )(depth_t, samples_t)


NGROUP = 4


@jax.jit
def _run(depth, weights, u):
    gsz = RAYS // NGROUP
    samples = [
        _sc_sample(depth[g * gsz:(g + 1) * gsz],
                   weights[g * gsz:(g + 1) * gsz], u)
        for g in range(NGROUP)
    ]
    outs = [
        _tc_sort(depth[g * gsz:(g + 1) * gsz].T, samples[g].T)
        for g in range(NGROUP)
    ]
    return jnp.concatenate(outs, axis=1).T


def kernel(depth_rays_values_coarse, coarse_weights, perturb):
    del perturb
    u = jnp.linspace(0.0, 1.0, NF, dtype=jnp.float32)
    return _run(depth_rays_values_coarse, coarse_weights, u)


# speedup vs baseline: 1.5744x; 1.0599x over previous
import functools

import jax
import jax.numpy as jnp
from jax import lax
from jax.experimental import pallas as pl
from jax.experimental.pallas import tpu as pltpu
from jax.experimental.pallas import tpu_sc as plsc

RAYS = 65536
NC = 64
NF = 128
NB = NC - 1
NW = NC - 2
NOUT = NC + NF
NSORT = 256
C = 128

NWORK = 32
RW = RAYS // NWORK
CH = 64
HLEN = 144
OFF = 8


def _sc_sample_kernel(depth_hbm, w_hbm, u_hbm, out_hbm,
                      dbuf, wbuf, sbuf, ubuf,
                      fv, mv, sl, al, ha, hb, dscr):
    widx = lax.axis_index("s") * 2 + lax.axis_index("c")
    pltpu.sync_copy(u_hbm, ubuf)
    iot = lax.iota(jnp.int32, 16)
    zero = jnp.zeros((16,), jnp.float32)
    fv[pl.ds(OFF + 64, 16)] = zero
    m_lo = iot >= 1
    m_hi = iot <= 14

    def ray_body(r, _):
        wv = [wbuf[r, pl.ds(16 * v, 16)] + 1e-5 for v in range(4)]
        wv[0] = jnp.where(m_lo, wv[0], zero)
        wv[3] = jnp.where(m_hi, wv[3], zero)
        sums = [jnp.sum(x) for x in wv]
        inv_v = 1.0 / jnp.full((16,), sums[0] + sums[1] + sums[2] + sums[3],
                               jnp.float32)
        carry = zero
        for v in range(4):
            pdfv = wv[v] * inv_v
            fv[pl.ds(OFF + 16 * v, 16)] = plsc.cumsum(pdfv) + carry
            carry = carry + jnp.full((16,), sums[v], jnp.float32) * inv_v

        d3 = dbuf[r, pl.ds(48, 16)]
        dscr[pl.ds(0, 16)] = d3
        for v in range(3):
            mv[pl.ds(OFF + 16 * v, 16)] = 0.5 * (
                dbuf[r, pl.ds(16 * v, 16)] + dbuf[r, pl.ds(16 * v + 1, 16)])
        mv[pl.ds(OFF + 48, 16)] = 0.5 * (d3 + dscr[pl.ds(1, 16)])

        for v in range(4):
            f0 = fv[pl.ds(OFF + 16 * v, 16)]
            fd = fv[pl.ds(OFF + 16 * v + 1, 16)] - f0
            den = jnp.where(fd < 1e-5, jnp.full((16,), 1.0, jnp.float32), fd)
            m0 = mv[pl.ds(OFF + 16 * v, 16)]
            s = (mv[pl.ds(OFF + 16 * v + 1, 16)] - m0) / den
            if v == 3:
                s = jnp.where(iot <= 13, s, zero)
            sl[pl.ds(OFF + 16 * v, 16)] = s
            al[pl.ds(OFF + 16 * v, 16)] = m0 - f0 * s

        for q in range(HLEN // 16):
            ha[pl.ds(16 * q, 16)] = zero
            hb[pl.ds(16 * q, 16)] = zero
        for v in range(4):
            f0 = fv[pl.ds(OFF + 16 * v, 16)]
            tf = f0 * 127.0
            ti = tf.astype(jnp.int32)
            t = jnp.minimum(ti + (ti.astype(jnp.float32) < tf), 128)
            da = al[pl.ds(OFF + 16 * v, 16)] - al[pl.ds(OFF + 16 * v - 1, 16)]
            ds_ = sl[pl.ds(OFF + 16 * v, 16)] - sl[pl.ds(OFF + 16 * v - 1, 16)]
            mask = m_lo if v == 0 else (m_hi if v == 3 else None)
            plsc.addupdate_scatter(ha, [t], da, mask=mask)
            plsc.addupdate_scatter(hb, [t], ds_, mask=mask)

        a0 = al[pl.ds(OFF, 16)][0]
        s0 = sl[pl.ds(OFF, 16)][0]
        ca = 0.0
        cb = 0.0
        for q in range(8):
            hav = ha[pl.ds(16 * q, 16)]
            hbv = hb[pl.ds(16 * q, 16)]
            acc_a = plsc.cumsum(hav) + (ca + a0)
            acc_b = plsc.cumsum(hbv) + (cb + s0)
            uq = ubuf[pl.ds(16 * q, 16)]
            sbuf[r, pl.ds(16 * q, 16)] = acc_a + uq * acc_b
            ca = ca + jnp.sum(hav)
            cb = cb + jnp.sum(hbv)
        return 0

    def chunk_body(g, _):
        base = widx * (depth_hbm.shape[0] // NWORK) + g * CH
        pltpu.sync_copy(depth_hbm.at[pl.ds(base, CH), :], dbuf)
        pltpu.sync_copy(w_hbm.at[pl.ds(base, CH), :], wbuf)
        lax.fori_loop(0, CH, ray_body, 0, unroll=2)
        pltpu.sync_copy(sbuf, out_hbm.at[pl.ds(base, CH), :])
        return 0

    lax.fori_loop(0, depth_hbm.shape[0] // NWORK // CH, chunk_body, 0,
                  unroll=False)


def _sc_sample(depth, weights, u):
    kfn = pl.kernel(
        _sc_sample_kernel,
        out_type=jax.ShapeDtypeStruct((depth.shape[0], NF), jnp.float32),
        mesh=plsc.VectorSubcoreMesh(core_axis_name="c", subcore_axis_name="s"),
        compiler_params=pltpu.CompilerParams(needs_layout_passes=False),
        scratch_types=[
            pltpu.VMEM((CH, NC), jnp.float32),
            pltpu.VMEM((CH, NC), jnp.float32),
            pltpu.VMEM((CH, NF), jnp.float32),
            pltpu.VMEM((NF,), jnp.float32),
            pltpu.VMEM((96,), jnp.float32),
            pltpu.VMEM((96,), jnp.float32),
            pltpu.VMEM((96,), jnp.float32),
            pltpu.VMEM((96,), jnp.float32),
            pltpu.VMEM((HLEN,), jnp.float32),
            pltpu.VMEM((HLEN,), jnp.float32),
            pltpu.VMEM((24,), jnp.float32),
        ],
    )
    return kfn(depth, weights, u)


def _substage(x, nrows, j, k, descending=False):
    m = nrows // (2 * j)
    y = x.reshape(m, 2 * j, C)
    a = y[:, :j, :]
    b = y[:, j:, :]
    lo = jnp.minimum(a, b)
    hi = jnp.maximum(a, b)
    if k >= nrows and not descending:
        na, nb = lo, hi
    elif k >= nrows:
        na, nb = hi, lo
    else:
        blk = jax.lax.broadcasted_iota(jnp.int32, (m, 1, C), 0)
        asc = ((blk * (2 * j)) & k) == 0
        if descending:
            asc = jnp.logical_not(asc)
        na = jnp.where(asc, lo, hi)
        nb = jnp.where(asc, hi, lo)
    return jnp.concatenate([na, nb], axis=1).reshape(nrows, C)


def _bitonic_sort(x, nrows, descending=False):
    k = 2
    while k <= nrows:
        j = k // 2
        while j >= 1:
            x = _substage(x, nrows, j, k, descending)
            j //= 2
        k *= 2
    return x


def _sort_body(d_ref, s_ref, o_ref):
    d = d_ref[...]
    samples = s_ref[...]
    s_sorted = _bitonic_sort(samples, NF, descending=False)
    d_sorted = _bitonic_sort(d, NC, descending=True)
    x = jnp.concatenate(
        [s_sorted, jnp.full((NSORT - NOUT, C), jnp.inf, jnp.float32),
         d_sorted], axis=0)
    j = NSORT // 2
    while j >= 1:
        x = _substage(x, NSORT, j, NSORT)
        j //= 2
    o_ref[...] = x[:NOUT, :]


def _tc_sort(depth_t, samples_t):
    nrays = depth_t.shape[1]
    return pl.pallas_call(
        _sort_body,
        grid=(nrays // C,),
        in_specs=[
            pl.BlockSpec((NC, C), lambda i: (0, i)),
            pl.BlockSpec((NF, C), lambda i: (0, i)),
        ],
        out_specs=pl.BlockSpec((NOUT, C), lambda i: (0, i)),
        out_shape=jax.ShapeDtypeStruct((NOUT, nrays), jnp.float32),
    )(depth_t, samples_t)


NGROUP = 8


@jax.jit
def _run(depth, weights, u):
    gsz = RAYS // NGROUP
    samples = [
        _sc_sample(depth[g * gsz:(g + 1) * gsz],
                   weights[g * gsz:(g + 1) * gsz], u)
        for g in range(NGROUP)
    ]
    outs = [
        _tc_sort(depth[g * gsz:(g + 1) * gsz].T, samples[g].T)
        for g in range(NGROUP)
    ]
    return jnp.concatenate(outs, axis=1).T


def kernel(depth_rays_values_coarse, coarse_weights, perturb):
    del perturb
    u = jnp.linspace(0.0, 1.0, NF, dtype=jnp.float32)
    return _run(depth_rays_values_coarse, coarse_weights, u)
